# baseline (speedup 1.0000x reference)
import functools

import jax
import jax.numpy as jnp
from jax import lax
from jax.experimental import pallas as pl
from jax.experimental.pallas import tpu as pltpu
from jax.experimental.pallas import tpu_sc as plsc

_TEMP = 0.07
_NC = 2
_NS = 16
_L = 16


def _sc_gather(table, anchors, pos_flat, neg_flat, N, B, P, NEG):
    NW = _NC * _NS
    nb = B // NW
    R = 8
    NG1 = nb // R
    NG2 = nb

    mesh = plsc.VectorSubcoreMesh(core_axis_name="c", subcore_axis_name="s")

    @functools.partial(
        pl.kernel,
        mesh=mesh,
        out_type=(
            jax.ShapeDtypeStruct((B * (P + 1) * NEG,), jnp.float32),
            jax.ShapeDtypeStruct((B * 2 * P,), jnp.float32),
        ),
        scratch_types=[
            pltpu.VMEM((nb,), jnp.int32),
            pltpu.VMEM((nb * P,), jnp.int32),
            pltpu.VMEM((nb * NEG,), jnp.int32),
            pltpu.VMEM_SHARED((_NS * 3 * R * N,), jnp.float32),
            pltpu.VMEM((R * NEG,), jnp.float32),
            pltpu.VMEM((_L,), jnp.float32),
            pltpu.VMEM((R * (NEG // _L + 1) * _L,), jnp.float32),
            pltpu.VMEM((nb * 2 * P,), jnp.float32),
            pltpu.SemaphoreType.DMA,
            pltpu.SemaphoreType.DMA,
            pltpu.SemaphoreType.DMA,
            pltpu.SemaphoreType.DMA,
        ],
    )
    def k(tab, anch_h, pos_h, neg_h, out1_h, out2_h,
          anch_v, pos_v, neg_v, sh_v, val_v, rp_v, pad_v, r9_v,
          sem_a, sem_b, sem_c, sem_e):
        sid = lax.axis_index("s")
        wid = sid * _NC + lax.axis_index("c")
        base = wid * nb
        sp0 = sid * (3 * R * N)

        pltpu.sync_copy(anch_h.at[pl.ds(base, nb)], anch_v)
        pltpu.sync_copy(pos_h.at[pl.ds(base * P, nb * P)], pos_v)
        pltpu.sync_copy(neg_h.at[pl.ds(base * NEG, nb * NEG)], neg_v)

        iota = lax.iota(jnp.int32, _L)
        zeros = jnp.zeros((_L,), jnp.int32)
        dnums = lax.GatherDimensionNumbers(
            offset_dims=(), collapsed_slice_dims=(0,), start_index_map=(0,))

        def lane_gather(vec, idx16):
            return lax.gather(vec, idx16[:, None], dnums, slice_sizes=(1,),
                              mode=lax.GatherScatterMode.PROMISE_IN_BOUNDS)

        def pos16_of(bl):
            pgrp = pl.multiple_of((bl // 2) * _L, _L)
            pv16 = pos_v[pl.ds(pgrp, _L)]
            return lane_gather(pv16, (iota % P) + (bl % 2) * P)

        def anch_splat(bl):
            agrp = pl.multiple_of((bl // _L) * _L, _L)
            av16 = anch_v[pl.ds(agrp, _L)]
            return lane_gather(av16, zeros + bl % _L)

        def fire1(g, buf, sem):
            av16 = anch_v[pl.ds(pl.multiple_of((g // 2) * _L, _L), _L)]
            rv = lane_gather(av16, (iota % R) + (g % 2) * R)
            for j in range(R):
                r = rv[j]
                pltpu.async_copy(tab.at[r],
                                 sh_v.at[pl.ds(sp0 + buf * R * N + j * N, N)],
                                 sem)

        def fire2(g, buf, sem):
            pv16 = pos_v[pl.ds(pl.multiple_of((g // 2) * _L, _L), _L)]
            rv = lane_gather(pv16, (iota % P) + (g % 2) * P)
            for j in range(P):
                r = rv[j]
                pltpu.async_copy(tab.at[r],
                                 sh_v.at[pl.ds(sp0 + buf * R * N + j * N, N)],
                                 sem)

        def drain(buf, sem):
            pltpu.make_async_copy(out1_h.at[pl.ds(0, R * N)],
                                  sh_v.at[pl.ds(sp0 + buf * R * N, R * N)],
                                  sem).wait()

        def drain_e(nvec):
            pltpu.make_async_copy(out1_h.at[pl.ds(0, nvec * _L)],
                                  pad_v.at[pl.ds(0, nvec * _L)],
                                  sem_e).wait()

        def extract1(g, buf):
            sb = sp0 + buf * R * N
            for j in range(R):
                bl = g * R + j
                pp = pos16_of(bl)
                pltpu.async_copy(sh_v.at[pp + (sb + j * N)],
                                 r9_v.at[pl.ds(bl * 2 * P, _L)], sem_e)
                for c in range(NEG // _L):
                    ng = neg_v[pl.ds(bl * NEG + c * _L, _L)]
                    pltpu.async_copy(
                        sh_v.at[ng + (sb + j * N)],
                        val_v.at[pl.ds(j * NEG + c * _L, _L)], sem_e)
            drain_e(R * (NEG // _L + 1))
            for j in range(R):
                off = ((base + g * R + j) * (P + 1) + P) * NEG
                pltpu.sync_copy(val_v.at[pl.ds(j * NEG, NEG)],
                                out1_h.at[pl.ds(off, NEG)])

        def extract2(g, buf):
            sb = sp0 + buf * R * N
            a_spl = anch_splat(g)
            for p in range(P):
                for c in range(NEG // _L):
                    ng = neg_v[pl.ds(g * NEG + c * _L, _L)]
                    pltpu.async_copy(
                        sh_v.at[ng + (sb + p * N)],
                        val_v.at[pl.ds(p * NEG + c * _L, _L)], sem_e)
            pltpu.async_copy(sh_v.at[(iota % P) * N + (sb + a_spl)],
                             rp_v, sem_e)
            drain_e(P * (NEG // _L) + 1)
            cur = r9_v[pl.ds(g * 2 * P, _L)]
            rp16 = rp_v[pl.ds(0, _L)]
            r9_v[pl.ds(g * 2 * P, _L)] = jnp.where(iota < P, cur, rp16)
            off = (base + g) * (P + 1) * NEG
            pltpu.sync_copy(val_v.at[pl.ds(0, P * NEG)],
                            out1_h.at[pl.ds(off, P * NEG)])

        def phase(ng, fire, extract):
            sems3 = (sem_a, sem_b, sem_c)
            fire(0, 0, sems3[0])
            fire(1, 1, sems3[1])

            def body(g, carry):
                for m in range(3):
                    prev2 = (m + 1) % 3

                    @pl.when(g % 3 == m)
                    def _(m=m, prev2=prev2):
                        fire(g, m, sems3[m])
                        drain(prev2, sems3[prev2])
                        extract(g - 2, prev2)

                return carry

            lax.fori_loop(2, ng, body, 0)
            for gg in (ng - 2, ng - 1):
                drain(gg % 3, sems3[gg % 3])
                extract(gg, gg % 3)

        phase(NG1, fire1, extract1)
        phase(NG2, fire2, extract2)
        pltpu.sync_copy(r9_v, out2_h.at[pl.ds(base * 2 * P, nb * 2 * P)])

    return k(table, anchors, pos_flat, neg_flat)


def _tc_loss(g1, g2, B, P, NEG, inv_count):
    BBLK = 256
    nsteps = B // BBLK

    def body(g1_ref, g2_ref, out_ref):
        i = pl.program_id(0)
        e = jnp.exp(g1_ref[...] / _TEMP)
        s_rev = jnp.sum(e[:, 0:P, :], axis=-1)
        s_fwd = jnp.sum(e[:, P, :], axis=-1)
        v = jnp.exp(g2_ref[...] / _TEMP)
        lane = lax.broadcasted_iota(jnp.int32, (BBLK, 2 * P), 1)
        s_sel = jnp.where(lane < P, s_fwd[:, None], 0.0)
        for p in range(P):
            s_sel = jnp.where(lane == P + p, s_rev[:, p:p + 1], s_sel)
        contrib = -jnp.log(v / (v + s_sel + 1e-10) + 1e-10)
        part = jnp.sum(contrib)

        @pl.when(i == 0)
        def _():
            out_ref[0, 0] = 0.0

        out_ref[0, 0] += part

        @pl.when(i == nsteps - 1)
        def _():
            out_ref[0, 0] = out_ref[0, 0] * inv_count

    return pl.pallas_call(
        body,
        grid=(nsteps,),
        in_specs=[
            pl.BlockSpec((BBLK, P + 1, NEG), lambda i: (i, 0, 0)),
            pl.BlockSpec((BBLK, 2 * P), lambda i: (i, 0)),
        ],
        out_specs=pl.BlockSpec(memory_space=pltpu.SMEM),
        out_shape=jax.ShapeDtypeStruct((1, 1), jnp.float32),
    )(g1, g2)


def kernel(ssms_list, anchors, positives, negatives, embeddings):
    num_ssms, N, _ = ssms_list.shape
    B, P = positives.shape
    NEG = negatives.shape[1]

    table = ssms_list.reshape(num_ssms * N, N)
    g1, g2 = _sc_gather(table, anchors, positives.reshape(-1),
                        negatives.reshape(-1), N, B, P, NEG)
    g1 = g1.reshape(B, P + 1, NEG)
    g2 = g2.reshape(B, 2 * P)
    inv_count = 1.0 / (2.0 * B * P * num_ssms)
    out = _tc_loss(g1, g2, B, P, NEG, inv_count)
    return out[0, 0]


# speedup vs baseline: 1.0101x; 1.0101x over previous
import functools

import jax
import jax.numpy as jnp
from jax import lax
from jax.experimental import pallas as pl
from jax.experimental.pallas import tpu as pltpu
from jax.experimental.pallas import tpu_sc as plsc

_TEMP = 0.07
_NC = 2
_NS = 16
_L = 16


def _sc_gather(table, anchors, pos_flat, neg_flat, N, B, P, NEG):
    NW = _NC * _NS
    nb = B // NW
    R = 8
    NG1 = nb // R
    NG2 = nb

    mesh = plsc.VectorSubcoreMesh(core_axis_name="c", subcore_axis_name="s")

    @functools.partial(
        pl.kernel,
        mesh=mesh,
        out_type=(
            jax.ShapeDtypeStruct((B * (P + 1) * NEG,), jnp.float32),
            jax.ShapeDtypeStruct((B * 2 * P,), jnp.float32),
        ),
        scratch_types=[
            pltpu.VMEM((nb,), jnp.int32),
            pltpu.VMEM((nb * P,), jnp.int32),
            pltpu.VMEM((nb * NEG,), jnp.int32),
            pltpu.VMEM_SHARED((_NS * 2 * R * N,), jnp.float32),
            pltpu.VMEM((R * NEG,), jnp.float32),
            pltpu.VMEM((_L,), jnp.float32),
            pltpu.VMEM((R * (NEG // _L + 1) * _L,), jnp.float32),
            pltpu.VMEM((nb * 2 * P,), jnp.float32),
            pltpu.SemaphoreType.DMA,
            pltpu.SemaphoreType.DMA,
            pltpu.SemaphoreType.DMA,
        ],
    )
    def k(tab, anch_h, pos_h, neg_h, out1_h, out2_h,
          anch_v, pos_v, neg_v, sh_v, val_v, rp_v, pad_v, r9_v,
          sem_a, sem_b, sem_e):
        sid = lax.axis_index("s")
        wid = sid * _NC + lax.axis_index("c")
        base = wid * nb
        sp0 = sid * (2 * R * N)

        pltpu.sync_copy(anch_h.at[pl.ds(base, nb)], anch_v)
        pltpu.sync_copy(pos_h.at[pl.ds(base * P, nb * P)], pos_v)
        pltpu.sync_copy(neg_h.at[pl.ds(base * NEG, nb * NEG)], neg_v)

        iota = lax.iota(jnp.int32, _L)
        zeros = jnp.zeros((_L,), jnp.int32)
        dnums = lax.GatherDimensionNumbers(
            offset_dims=(), collapsed_slice_dims=(0,), start_index_map=(0,))

        def lane_gather(vec, idx16):
            return lax.gather(vec, idx16[:, None], dnums, slice_sizes=(1,),
                              mode=lax.GatherScatterMode.PROMISE_IN_BOUNDS)

        def pos16_of(bl):
            pgrp = pl.multiple_of((bl // 2) * _L, _L)
            pv16 = pos_v[pl.ds(pgrp, _L)]
            return lane_gather(pv16, (iota % P) + (bl % 2) * P)

        def anch_splat(bl):
            agrp = pl.multiple_of((bl // _L) * _L, _L)
            av16 = anch_v[pl.ds(agrp, _L)]
            return lane_gather(av16, zeros + bl % _L)

        def fire1(g, buf, sem, half):
            av16 = anch_v[pl.ds(pl.multiple_of((g // 2) * _L, _L), _L)]
            for j in range(R):
                r = av16[half * R + j]
                pltpu.async_copy(tab.at[r],
                                 sh_v.at[pl.ds(sp0 + buf * R * N + j * N, N)],
                                 sem)

        def fire2(g, buf, sem, half):
            pv16 = pos_v[pl.ds(pl.multiple_of((g // 2) * _L, _L), _L)]
            for j in range(P):
                r = pv16[half * P + j]
                pltpu.async_copy(tab.at[r],
                                 sh_v.at[pl.ds(sp0 + buf * R * N + j * N, N)],
                                 sem)

        def drain(buf, sem):
            pltpu.make_async_copy(out1_h.at[pl.ds(0, R * N)],
                                  sh_v.at[pl.ds(sp0 + buf * R * N, R * N)],
                                  sem).wait()

        def drain_e(nvec):
            pltpu.make_async_copy(out1_h.at[pl.ds(0, nvec * _L)],
                                  pad_v.at[pl.ds(0, nvec * _L)],
                                  sem_e).wait()

        def extract1(g, buf):
            sb = sp0 + buf * R * N
            for j in range(R):
                bl = g * R + j
                pp = pos16_of(bl)
                pltpu.async_copy(sh_v.at[pp + (sb + j * N)],
                                 r9_v.at[pl.ds(bl * 2 * P, _L)], sem_e)
                for c in range(NEG // _L):
                    ng = neg_v[pl.ds(bl * NEG + c * _L, _L)]
                    pltpu.async_copy(
                        sh_v.at[ng + (sb + j * N)],
                        val_v.at[pl.ds(j * NEG + c * _L, _L)], sem_e)
            drain_e(R * (NEG // _L + 1))
            for j in range(R):
                off = ((base + g * R + j) * (P + 1) + P) * NEG
                pltpu.sync_copy(val_v.at[pl.ds(j * NEG, NEG)],
                                out1_h.at[pl.ds(off, NEG)])

        def extract2(g, buf):
            sb = sp0 + buf * R * N
            a_spl = anch_splat(g)
            for p in range(P):
                for c in range(NEG // _L):
                    ng = neg_v[pl.ds(g * NEG + c * _L, _L)]
                    pltpu.async_copy(
                        sh_v.at[ng + (sb + p * N)],
                        val_v.at[pl.ds(p * NEG + c * _L, _L)], sem_e)
            pltpu.async_copy(sh_v.at[(iota % P) * N + (sb + a_spl)],
                             rp_v, sem_e)
            drain_e(P * (NEG // _L) + 1)
            cur = r9_v[pl.ds(g * 2 * P, _L)]
            rp16 = rp_v[pl.ds(0, _L)]
            r9_v[pl.ds(g * 2 * P, _L)] = jnp.where(iota < P, cur, rp16)
            off = (base + g) * (P + 1) * NEG
            pltpu.sync_copy(val_v.at[pl.ds(0, P * NEG)],
                            out1_h.at[pl.ds(off, P * NEG)])

        def phase(ng, fire, extract):
            fire(0, 0, sem_a, 0)

            def body(g, carry):
                @pl.when(g % 2 == 1)
                def _():
                    fire(g, 1, sem_b, 1)
                    drain(0, sem_a)
                    extract(g - 1, 0)

                @pl.when(g % 2 == 0)
                def _():
                    fire(g, 0, sem_a, 0)
                    drain(1, sem_b)
                    extract(g - 1, 1)

                return carry

            lax.fori_loop(1, ng, body, 0)
            lastbuf, lastsem = (1, sem_b) if ng % 2 == 0 else (0, sem_a)
            drain(lastbuf, lastsem)
            extract(ng - 1, lastbuf)

        phase(NG1, fire1, extract1)
        phase(NG2, fire2, extract2)
        pltpu.sync_copy(r9_v, out2_h.at[pl.ds(base * 2 * P, nb * 2 * P)])

    return k(table, anchors, pos_flat, neg_flat)


def _tc_loss(g1, g2, B, P, NEG, inv_count):
    BBLK = 256
    nsteps = B // BBLK

    def body(g1_ref, g2_ref, out_ref):
        i = pl.program_id(0)
        e = jnp.exp(g1_ref[...] / _TEMP)
        s_rev = jnp.sum(e[:, 0:P, :], axis=-1)
        s_fwd = jnp.sum(e[:, P, :], axis=-1)
        v = jnp.exp(g2_ref[...] / _TEMP)
        lane = lax.broadcasted_iota(jnp.int32, (BBLK, 2 * P), 1)
        s_sel = jnp.where(lane < P, s_fwd[:, None], 0.0)
        for p in range(P):
            s_sel = jnp.where(lane == P + p, s_rev[:, p:p + 1], s_sel)
        contrib = -jnp.log(v / (v + s_sel + 1e-10) + 1e-10)
        part = jnp.sum(contrib)

        @pl.when(i == 0)
        def _():
            out_ref[0, 0] = 0.0

        out_ref[0, 0] += part

        @pl.when(i == nsteps - 1)
        def _():
            out_ref[0, 0] = out_ref[0, 0] * inv_count

    return pl.pallas_call(
        body,
        grid=(nsteps,),
        in_specs=[
            pl.BlockSpec((BBLK, P + 1, NEG), lambda i: (i, 0, 0)),
            pl.BlockSpec((BBLK, 2 * P), lambda i: (i, 0)),
        ],
        out_specs=pl.BlockSpec(memory_space=pltpu.SMEM),
        out_shape=jax.ShapeDtypeStruct((1, 1), jnp.float32),
    )(g1, g2)


def kernel(ssms_list, anchors, positives, negatives, embeddings):
    num_ssms, N, _ = ssms_list.shape
    B, P = positives.shape
    NEG = negatives.shape[1]

    table = ssms_list.reshape(num_ssms * N, N)
    g1, g2 = _sc_gather(table, anchors, positives.reshape(-1),
                        negatives.reshape(-1), N, B, P, NEG)
    g1 = g1.reshape(B, P + 1, NEG)
    g2 = g2.reshape(B, 2 * P)
    inv_count = 1.0 / (2.0 * B * P * num_ssms)
    out = _tc_loss(g1, g2, B, P, NEG, inv_count)
    return out[0, 0]
